# baseline (speedup 1.0000x reference)
import jax
import jax.numpy as jnp
from jax import lax
from jax.experimental import pallas as pl
from jax.experimental.pallas import tpu as pltpu
from jax.experimental.pallas import tpu_sc as plsc

_L = 16
_NW = 32
_WCH = 8

_SORT8 = [(0, 1), (2, 3), (4, 5), (6, 7),
          (0, 2), (1, 3), (4, 6), (5, 7),
          (1, 2), (5, 6), (0, 4), (3, 7),
          (1, 5), (2, 6),
          (1, 4), (3, 6),
          (2, 4), (3, 5),
          (3, 4)]

_BITONIC8 = [(0, 4), (1, 5), (2, 6), (3, 7),
             (0, 2), (1, 3), (4, 6), (5, 7),
             (0, 1), (2, 3), (4, 5), (6, 7)]


def _sort8(v):
    v = list(v)
    for a, b in _SORT8:
        hi = jnp.maximum(v[a], v[b])
        lo = jnp.minimum(v[a], v[b])
        v[a] = hi
        v[b] = lo
    return v


def _merge8(a, b):
    m = [jnp.maximum(a[j], b[7 - j]) for j in range(8)]
    for p, q in _BITONIC8:
        hi = jnp.maximum(m[p], m[q])
        lo = jnp.minimum(m[p], m[q])
        m[p] = hi
        m[q] = lo
    return m


def _merge8_top(a, b):
    return [jnp.maximum(a[j], b[7 - j]) for j in range(8)]


def _xmerge(a, b):
    return jnp.sort(jnp.maximum(a, jnp.flip(b)))


def _make_sc_call(B, C, D, H, W):
    NT = B * H
    NTW = NT // _NW
    NCH = W // _WCH
    NQ = NTW * NCH
    NG = C // _L
    CD = C * D
    assert NT % _NW == 0 and W % _WCH == 0 and C % _L == 0 and D == 16

    def body(x_hbm, tk_hbm, mn_hbm, buf, stg_tk, stg_mn, sem):
        nc = plsc.get_sparse_core_info().num_cores
        wid = lax.axis_index("s") * nc + lax.axis_index("c")

        def chunk_src(q):
            tid = wid + 32 * (q // NCH)
            c = q % NCH
            b = tid // H
            h = tid % H
            ws = pl.multiple_of(c * _WCH, _WCH)
            return x_hbm.at[b, :, h, pl.ds(ws, _WCH), :], b, h, c

        src0, _, _, _ = chunk_src(0)
        pltpu.make_async_copy(src0, buf.at[0], sem).start()
        src1, _, _, _ = chunk_src(1)
        pltpu.make_async_copy(src1, buf.at[1], sem).start()

        def q_body(q, carry):
            restk, resmn = carry
            par = lax.rem(q, 3)
            src, b, h, c = chunk_src(q)
            pltpu.make_async_copy(src, buf.at[par], sem).wait()

            @pl.when(q < NQ - 2)
            def _():
                srcn, _, _, _ = chunk_src(q + 2)
                pltpu.make_async_copy(srcn,
                                      buf.at[lax.rem(q + 2, 3)],
                                      sem).start()

            iota = lax.broadcasted_iota(jnp.int32, (_L,), 0)

            def bcast_sum(vv):
                cs = plsc.cumsum(vv)
                head = jnp.where(iota == 0, jnp.flip(cs), 0.0)
                return plsc.cumsum(head)

            def one_px(pp):
                groups = []
                tot = jnp.zeros((_L,), jnp.float32)
                for g in range(NG):
                    v = [buf[par, d, pp, pl.ds(g * _L, _L)]
                         for d in range(D)]
                    s01 = (v[0] + v[1]) + (v[2] + v[3])
                    s23 = (v[4] + v[5]) + (v[6] + v[7])
                    s45 = (v[8] + v[9]) + (v[10] + v[11])
                    s67 = (v[12] + v[13]) + (v[14] + v[15])
                    tot = tot + ((s01 + s23) + (s45 + s67))
                    groups.append(_merge8(_sort8(v[0:8]), _sort8(v[8:16])))
                m01 = _merge8(groups[0], groups[1])
                m23 = _merge8(groups[2], groups[3])
                mall = _merge8_top(m01, m23)
                ss = [jnp.sort(mall[j]) for j in range(8)]
                r = _xmerge(_xmerge(_xmerge(ss[0], ss[1]),
                                    _xmerge(ss[2], ss[3])),
                            _xmerge(_xmerge(ss[4], ss[5]),
                                    _xmerge(ss[6], ss[7])))
                tk_b = bcast_sum(jnp.where(iota >= 8, r, 0.0))
                mn_b = bcast_sum(tot) * (1.0 / CD)
                return tk_b, mn_b

            def px_body(pe, cr2):
                restk, resmn = cr2
                for u in range(2):
                    pp = 2 * pe + u
                    tk_b, mn_b = one_px(pp)
                    slot = lax.rem(_WCH * c + pp, _L)
                    restk = jnp.where(iota == slot, tk_b, restk)
                    resmn = jnp.where(iota == slot, mn_b, resmn)
                return (restk, resmn)

            restk, resmn = lax.fori_loop(0, _WCH // 2, px_body,
                                         (restk, resmn))

            @pl.when((lax.rem(c, 2) == 1) | (c == NCH - 1))
            def _():
                off = (c // 2) * _L
                stg_tk[0, pl.ds(off, _L)] = restk
                stg_mn[0, pl.ds(off, _L)] = resmn

            @pl.when(c == NCH - 1)
            def _():
                pltpu.sync_copy(stg_tk, tk_hbm.at[b, h])
                pltpu.sync_copy(stg_mn, mn_hbm.at[b, h])

            return (restk, resmn)

        z = jnp.zeros((_L,), jnp.float32)
        lax.fori_loop(0, NQ, q_body, (z, z))

    mesh = plsc.VectorSubcoreMesh(core_axis_name="c", subcore_axis_name="s")
    return pl.kernel(
        body,
        out_type=[jax.ShapeDtypeStruct((B, H, 8, 64), jnp.float32),
                  jax.ShapeDtypeStruct((B, H, 8, 64), jnp.float32)],
        mesh=mesh,
        compiler_params=pltpu.CompilerParams(needs_layout_passes=False),
        scratch_types=[pltpu.VMEM((3, D, _WCH, C), jnp.float32),
                       pltpu.VMEM((8, 64), jnp.float32),
                       pltpu.VMEM((8, 64), jnp.float32),
                       pltpu.SemaphoreType.DMA],
    )


def kernel(x):
    b, c, d, h, w = x.shape
    xt = jnp.transpose(x, (0, 2, 3, 4, 1))
    tk, mn = _make_sc_call(b, c, d, h, w)(xt)
    tk = tk[:, :, 0, :w].reshape(b, 1, 1, h, w)
    mn = mn[:, :, 0, :w].reshape(b, 1, 1, h, w)
    return (tk, mn)
